# baseline (speedup 1.0000x reference)
import jax
import jax.numpy as jnp
from jax.experimental import pallas as pl
from jax.experimental.pallas import tpu as pltpu

_HOP = 320
_CONTRACT_LAST = (((1,), (1,)), ((), ()))


def _vq_body(frames_ref, W1_ref, b1_ref, W2_ref, b2_ref, cb_ref, out_ref,
             cb2_ref):
    @pl.when(pl.program_id(0) == 0)
    def _():
        cb0 = cb_ref[...]
        cb2_ref[...] = jnp.sum(cb0 * cb0, axis=1, keepdims=True).T

    f = frames_ref[...]
    h = jnp.tanh(
        jnp.dot(f, W1_ref[...], preferred_element_type=jnp.float32)
        + b1_ref[...])
    c = jnp.tanh(
        jnp.dot(h, W2_ref[...], preferred_element_type=jnp.float32)
        + b2_ref[...])
    z2 = jnp.sum(c * c, axis=1, keepdims=True)
    cross = jax.lax.dot_general(c, cb_ref[...], _CONTRACT_LAST,
                                preferred_element_type=jnp.float32)
    s = z2 - 2.0 * cross + cb2_ref[...]
    out_ref[0, 0, :] = jnp.argmin(s, axis=1).astype(jnp.int32)


def kernel(x, W1, b1, W2, b2, codebook):
    B = x.shape[0]
    if x.ndim == 3 and x.shape[-1] == 1:
        x = x[..., 0]
    T = x.shape[1] // _HOP
    M = B * T
    G, K, Dg = codebook.shape
    D = W2.shape[1]
    frames = x[:, : T * _HOP].reshape(M, _HOP)

    MT = 1600
    grid = M // MT
    out = pl.pallas_call(
        _vq_body,
        grid=(grid,),
        in_specs=[
            pl.BlockSpec((MT, _HOP), lambda i: (i, 0)),
            pl.BlockSpec((_HOP, D), lambda i: (0, 0)),
            pl.BlockSpec((1, D), lambda i: (0, 0)),
            pl.BlockSpec((D, D), lambda i: (0, 0)),
            pl.BlockSpec((1, D), lambda i: (0, 0)),
            pl.BlockSpec((K, Dg), lambda i: (0, 0)),
        ],
        out_specs=pl.BlockSpec((1, 1, MT), lambda i: (i, 0, 0)),
        out_shape=jax.ShapeDtypeStruct((grid, 1, MT), jnp.int32),
        scratch_shapes=[pltpu.VMEM((1, K), jnp.float32)],
    )(frames, W1, b1[None], W2, b2[None], codebook[0])
    return out.reshape(B, T, G).astype(jnp.int32)


# speedup vs baseline: 1.0582x; 1.0582x over previous
import jax
import jax.numpy as jnp
from jax.experimental import pallas as pl
from jax.experimental.pallas import tpu as pltpu

_HOP = 320
_CONTRACT_LAST = (((1,), (1,)), ((), ()))


_CHUNK = 400


def _vq_body(frames_ref, W1_ref, b1_ref, W2_ref, b2_ref, cb_ref, out_ref,
             cb2_ref):
    @pl.when(pl.program_id(0) == 0)
    def _():
        cb0 = cb_ref[...]
        cb2_ref[...] = jnp.sum(cb0 * cb0, axis=1, keepdims=True).T

    W1 = W1_ref[...]
    W2 = W2_ref[...]
    b1 = b1_ref[...]
    b2 = b2_ref[...]
    cb = cb_ref[...]
    cb2 = cb2_ref[...]
    mt = frames_ref.shape[0]
    for j in range(0, mt, _CHUNK):
        f = frames_ref[pl.ds(j, _CHUNK), :]
        h = jnp.tanh(
            jnp.dot(f, W1, preferred_element_type=jnp.float32) + b1)
        c = jnp.tanh(
            jnp.dot(h, W2, preferred_element_type=jnp.float32) + b2)
        z2 = jnp.sum(c * c, axis=1, keepdims=True)
        cross = jax.lax.dot_general(c, cb, _CONTRACT_LAST,
                                    preferred_element_type=jnp.float32)
        s = z2 - 2.0 * cross + cb2
        out_ref[0, 0, pl.ds(j, _CHUNK)] = jnp.argmin(s, axis=1).astype(
            jnp.int32)


def kernel(x, W1, b1, W2, b2, codebook):
    B = x.shape[0]
    if x.ndim == 3 and x.shape[-1] == 1:
        x = x[..., 0]
    T = x.shape[1] // _HOP
    M = B * T
    G, K, Dg = codebook.shape
    D = W2.shape[1]
    frames = x[:, : T * _HOP].reshape(M, _HOP)

    MT = 1600
    grid = M // MT
    out = pl.pallas_call(
        _vq_body,
        grid=(grid,),
        in_specs=[
            pl.BlockSpec((MT, _HOP), lambda i: (i, 0)),
            pl.BlockSpec((_HOP, D), lambda i: (0, 0)),
            pl.BlockSpec((1, D), lambda i: (0, 0)),
            pl.BlockSpec((D, D), lambda i: (0, 0)),
            pl.BlockSpec((1, D), lambda i: (0, 0)),
            pl.BlockSpec((K, Dg), lambda i: (0, 0)),
        ],
        out_specs=pl.BlockSpec((1, 1, MT), lambda i: (i, 0, 0)),
        out_shape=jax.ShapeDtypeStruct((grid, 1, MT), jnp.int32),
        scratch_shapes=[pltpu.VMEM((1, K), jnp.float32)],
    )(frames, W1, b1[None], W2, b2[None], codebook[0])
    return out.reshape(B, T, G).astype(jnp.int32)
